# baseline (speedup 1.0000x reference)
import dataclasses
import functools

import jax
import jax.numpy as jnp
from jax import lax
from jax.experimental import pallas as pl
from jax.experimental.pallas import tpu as pltpu
from jax.experimental.pallas import tpu_sc as plsc

_B = 4096
_D_IN = 1024
_D_H = 1024
_SW = 16384

_NC = 2
_NS = 16
_NW = _NC * _NS
_BPW = _B // _NW

_BM = 512


def _sc_gather(table, idx):
  mesh = plsc.VectorSubcoreMesh(core_axis_name="c", subcore_axis_name="s")
  cp = pltpu.CompilerParams()
  if "needs_layout_passes" in pltpu.CompilerParams.__dataclass_fields__:
    cp = dataclasses.replace(cp, needs_layout_passes=False)

  @functools.partial(
      pl.kernel,
      mesh=mesh,
      compiler_params=cp,
      out_type=jax.ShapeDtypeStruct((_B,), jnp.float32),
      scratch_types=[
          pltpu.VMEM((_SW,), jnp.float32),
          pltpu.VMEM((_BPW,), jnp.int32),
          pltpu.VMEM((_BPW,), jnp.float32),
          pltpu.SemaphoreType.DMA,
      ],
  )
  def k(table_hbm, idx_hbm, out_hbm, table_v, idx_v, out_v, sem):
    wid = lax.axis_index("s") * _NC + lax.axis_index("c")
    base = wid * _BPW
    cp = pltpu.async_copy(table_hbm, table_v, sem)
    pltpu.sync_copy(idx_hbm.at[pl.ds(base, _BPW)], idx_v)
    cp.wait()

    @pl.loop(0, _BPW, step=16)
    def _(j):
      iv = idx_v[pl.ds(j, 16)]
      out_v[pl.ds(j, 16)] = plsc.load_gather(table_v, [iv])

    pltpu.sync_copy(out_v, out_hbm.at[pl.ds(base, _BPW)])

  return k(table, idx)


def _tc_body(x_ref, y_ref, sw_ref, w1_ref, b1_ref, w2_ref, b2_ref, out_ref):
  xb = x_ref[...].astype(jnp.bfloat16)
  w1b = w1_ref[...].astype(jnp.bfloat16)
  h = jnp.dot(xb, w1b, preferred_element_type=jnp.float32)
  h = jnp.maximum(h + b1_ref[...], 0.0)
  pred = jnp.sum(h * w2_ref[...], axis=1, keepdims=True)
  e = pred + b2_ref[...] - y_ref[...]
  partial = jnp.sum(e * e * sw_ref[...], axis=(0, 1), keepdims=True) * (1.0 / _B)

  @pl.when(pl.program_id(0) == 0)
  def _():
    out_ref[...] = jnp.zeros_like(out_ref)

  out_ref[...] += partial


def _tc_loss(x, y, swg, w1, b1, w2, b2):
  grid = (_B // _BM,)
  out = pl.pallas_call(
      _tc_body,
      grid=grid,
      in_specs=[
          pl.BlockSpec((_BM, _D_IN), lambda i: (i, 0)),
          pl.BlockSpec((_BM, 1), lambda i: (i, 0)),
          pl.BlockSpec((_BM, 1), lambda i: (i, 0)),
          pl.BlockSpec((_D_IN, _D_H), lambda i: (0, 0)),
          pl.BlockSpec((1, _D_H), lambda i: (0, 0)),
          pl.BlockSpec((1, _D_H), lambda i: (0, 0)),
          pl.BlockSpec((1, 1), lambda i: (0, 0)),
      ],
      out_specs=pl.BlockSpec((1, 1), lambda i: (0, 0)),
      out_shape=jax.ShapeDtypeStruct((1, 1), jnp.float32),
      compiler_params=pltpu.CompilerParams(
          dimension_semantics=("arbitrary",),
      ),
  )(x, y, swg, w1, b1, w2, b2)
  return out.reshape(())


def kernel(x, y, index, W1, b1, W2, b2, sample_weights):
  idx = index.reshape(_B)
  swg = _sc_gather(sample_weights, idx).reshape(_B, 1)
  return _tc_loss(
      x,
      y,
      swg,
      W1,
      b1.reshape(1, _D_H),
      W2.reshape(1, _D_H),
      b2.reshape(1, 1),
  )
